# baseline (speedup 1.0000x reference)
import functools

import jax
import jax.numpy as jnp
from jax import lax
from jax.experimental import pallas as pl
from jax.experimental.pallas import tpu as pltpu
from jax.experimental.pallas import tpu_sc as plsc

B = 4
MD = 256
NPTS = 16384
H = 4
F = 64
GS = 16
G = GS ** 3

NB = 2048
OC = H * (F + 3)
OCP = 16 + MD

_PREC = lax.Precision.HIGHEST


def _stats_body(x_ref, s_ref, sx_ref):
    b = pl.program_id(0)
    j = pl.program_id(1)

    @pl.when((b == 0) & (j == 0))
    def _():
        s_ref[...] = jnp.zeros_like(s_ref)
        sx_ref[...] = jnp.zeros_like(sx_ref)

    xb = x_ref[0]
    s_ref[...] += lax.dot_general(xb, xb, (((1,), (1,)), ((), ())),
                                  precision=_PREC,
                                  preferred_element_type=jnp.float32)
    sx_ref[...] += jnp.sum(xb, axis=1, keepdims=True)


def _stats(x):
    return pl.pallas_call(
        _stats_body,
        grid=(B, NPTS // NB),
        in_specs=[pl.BlockSpec((1, MD, NB), lambda b, j: (b, 0, j))],
        out_specs=[
            pl.BlockSpec((MD, MD), lambda b, j: (0, 0)),
            pl.BlockSpec((MD, 1), lambda b, j: (0, 0)),
        ],
        out_shape=[
            jax.ShapeDtypeStruct((MD, MD), jnp.float32),
            jax.ShapeDtypeStruct((MD, 1), jnp.float32),
        ],
    )(x)


def _fold_body(w_ref, s_ref, sx_ref, m_ref, sg_ref):
    w = w_ref[...]
    s = s_ref[...]
    sx = sx_ref[...]
    inv = 1.0 / (B * NPTS)
    m = jnp.dot(w, sx, precision=_PREC,
                preferred_element_type=jnp.float32) * inv
    ws = jnp.dot(w, s, precision=_PREC,
                 preferred_element_type=jnp.float32)
    e2 = jnp.sum(ws * w, axis=1, keepdims=True) * inv
    var = e2 - m * m
    m_ref[...] = m
    sg_ref[...] = jnp.sqrt(var + 1e-5)


def _fold(w_pad, s, sx):
    return pl.pallas_call(
        _fold_body,
        out_shape=[
            jax.ShapeDtypeStruct((OCP, 1), jnp.float32),
            jax.ShapeDtypeStruct((OCP, 1), jnp.float32),
        ],
    )(w_pad, s, sx)


def _proj_body(x_ref, o_ref, wk_ref, wv_ref, mk_ref, sk_ref, gk_ref, bk_ref,
               mv_ref, sv_ref, gv_ref, bv_ref, t_ref,
               val_ref, w8_ref, idx_ref):
    xb = x_ref[0]
    kv_v = jnp.dot(wv_ref[...], xb, preferred_element_type=jnp.float32)
    val_ref[0] = ((kv_v - mv_ref[...]) / sv_ref[...]) * gv_ref[...] + bv_ref[...]
    kv_k = jnp.dot(wk_ref[...], xb, preferred_element_type=jnp.float32)
    k16 = ((kv_k - mk_ref[...]) / sk_ref[...]) * gk_ref[...] + bk_ref[...]
    o = o_ref[0]

    def _bf(v):
        return v.astype(jnp.bfloat16).astype(jnp.float32)

    half = 0.5 * (GS - 1)
    for h in range(H):
        p = [_bf(o[d:d + 1] + k16[3 * h + d:3 * h + d + 1]) for d in range(3)]
        bs, fr = [], []
        for i in range(3):
            key = (_bf(t_ref[0, h * 9 + i * 3 + 0]) * p[0]
                   + _bf(t_ref[0, h * 9 + i * 3 + 1]) * p[1]
                   + _bf(t_ref[0, h * 9 + i * 3 + 2]) * p[2])
            pos = (jnp.tanh(key) + 1.0) * half
            base = jnp.clip(jnp.floor(pos), 0.0, GS - 2.0)
            bs.append(base)
            fr.append(pos - base)
        idx0 = (bs[0].astype(jnp.int32) * (GS * GS)
                + bs[1].astype(jnp.int32) * GS
                + bs[2].astype(jnp.int32))
        idx_ref[0, h] = idx0[0]
        gx = (1.0 - fr[0], fr[0])
        gy = (1.0 - fr[1], fr[1])
        gz = (1.0 - fr[2], fr[2])
        c = 0
        for cx in range(2):
            for cy in range(2):
                for cz in range(2):
                    w8_ref[0, h * 8 + c] = (gx[cx] * gy[cy] * gz[cz])[0]
                    c += 1


def _project(x, o_pad, wk, wv, mk, sk, gk, bk, mv, sv, gv, bv, t_flat):
    return pl.pallas_call(
        _proj_body,
        grid=(B, NPTS // NB),
        in_specs=[
            pl.BlockSpec((1, MD, NB), lambda b, j: (b, 0, j)),
            pl.BlockSpec((1, 8, NB), lambda b, j: (b, 0, j)),
            pl.BlockSpec((16, MD), lambda b, j: (0, 0)),
            pl.BlockSpec((MD, MD), lambda b, j: (0, 0)),
        ] + [pl.BlockSpec((16, 1), lambda b, j: (0, 0))] * 4
          + [pl.BlockSpec((MD, 1), lambda b, j: (0, 0))] * 4
          + [pl.BlockSpec((1, 64), lambda b, j: (0, 0))],
        out_specs=[
            pl.BlockSpec((1, MD, NB), lambda b, j: (b, 0, j)),
            pl.BlockSpec((1, 32, NB), lambda b, j: (b, 0, j)),
            pl.BlockSpec((1, 8, NB), lambda b, j: (b, 0, j)),
        ],
        out_shape=[
            jax.ShapeDtypeStruct((B, MD, NPTS), jnp.float32),
            jax.ShapeDtypeStruct((B, 32, NPTS), jnp.float32),
            jax.ShapeDtypeStruct((B, 8, NPTS), jnp.int32),
        ],
    )(x, o_pad, wk, wv, mk, sk, gk, bk, mv, sv, gv, bv, t_flat)


_NW = 32
_P = 512
_NCHUNK = NPTS // _P
_OFFS = (0, 1, GS, GS + 1, GS * GS, GS * GS + 1, GS * GS + GS, GS * GS + GS + 1)


def _scatter_body(vals_hbm, w8_hbm, idx_hbm, out_hbm, table,
                  vb0, wb0, ib0, vb1, wb1, ib1, sem0, sem1):
    wid = lax.axis_index("s") * 2 + lax.axis_index("c")
    bufs = ((vb0, wb0, ib0, sem0), (vb1, wb1, ib1, sem1))

    for k in range(2):
        t = wid * 2 + k
        r = t // 4
        q = t % 4
        bb = r // H
        hh = r % H

        @plsc.parallel_loop(0, 16 * G // 16)
        def _zero(i):
            table[pl.ds(i * 16, 16)] = jnp.zeros((16,), jnp.float32)

        def _slices(ch, j):
            p0 = ch * _P
            vb, wb, ib, sem = bufs[j]
            return (
                (vals_hbm.at[r, pl.ds(q * 16, 16), pl.ds(p0, _P)], vb, sem),
                (w8_hbm.at[bb, pl.ds(hh * 8, 8), pl.ds(p0, _P)], wb, sem),
                (idx_hbm.at[bb, hh, pl.ds(p0, _P)], ib, sem),
            )

        def _issue(ch, j):
            for src, dst, sem in _slices(ch, j):
                pltpu.async_copy(src, dst, sem)

        def _wait(ch, j):
            for src, dst, sem in _slices(ch, j):
                pltpu.make_async_copy(src, dst, sem).wait()

        def _compute(j):
            vb, wb, ib, _ = bufs[j]

            @plsc.parallel_loop(0, _P // 16, unroll=2)
            def _group(g):
                s = g * 16
                idxv = ib[pl.ds(s, 16)]
                wcs = [wb[c, pl.ds(s, 16)] for c in range(8)]
                acs = [idxv + _OFFS[c] for c in range(8)]
                for f in range(16):
                    v = vb[f, pl.ds(s, 16)]
                    fG = f * G
                    for c in range(8):
                        plsc.addupdate_scatter(table, [acs[c] + fG],
                                               v * wcs[c])

        _issue(0, 0)
        _issue(1, 1)

        def _pair(cp, _):
            ch0 = cp * 2
            _wait(ch0, 0)
            _compute(0)

            @pl.when(ch0 + 2 < _NCHUNK)
            def _():
                _issue(ch0 + 2, 0)

            _wait(ch0 + 1, 1)
            _compute(1)

            @pl.when(ch0 + 3 < _NCHUNK)
            def _():
                _issue(ch0 + 3, 1)
            return 0

        lax.fori_loop(0, _NCHUNK // 2, _pair, 0)

        for f in range(16):
            pltpu.sync_copy(table.at[pl.ds(f * G, G)],
                            out_hbm.at[r, q * 16 + f, :])


def _scatter(vals16, w8, idx0):
    mesh = plsc.VectorSubcoreMesh(core_axis_name="c", subcore_axis_name="s")
    f = pl.kernel(
        _scatter_body,
        out_type=jax.ShapeDtypeStruct((B * H, F, G), jnp.float32),
        mesh=mesh,
        compiler_params=pltpu.CompilerParams(needs_layout_passes=False),
        scratch_types=[
            pltpu.VMEM((16 * G,), jnp.float32),
            pltpu.VMEM((16, _P), jnp.float32),
            pltpu.VMEM((8, _P), jnp.float32),
            pltpu.VMEM((_P,), jnp.int32),
            pltpu.VMEM((16, _P), jnp.float32),
            pltpu.VMEM((8, _P), jnp.float32),
            pltpu.VMEM((_P,), jnp.int32),
            pltpu.SemaphoreType.DMA,
            pltpu.SemaphoreType.DMA,
        ],
    )
    return f(vals16, w8, idx0)


def kernel(input_tensor, original_points, W_kv, gamma_k, beta_k,
           gamma_v, beta_v, T):
    x = input_tensor.astype(jnp.float32)

    w_pad = jnp.concatenate(
        [W_kv[:H * 3], jnp.zeros((4, MD), jnp.float32), W_kv[H * 3:]], axis=0)
    gam_pad = jnp.concatenate(
        [gamma_k, jnp.zeros((4,), jnp.float32), gamma_v])[:, None]
    bet_pad = jnp.concatenate(
        [beta_k, jnp.zeros((4,), jnp.float32), beta_v])[:, None]

    s, sx = _stats(x)
    m, sg = _fold(w_pad, s, sx)

    o_pad = jnp.concatenate(
        [original_points, jnp.zeros((B, 5, NPTS), jnp.float32)], axis=1)
    t_flat = jnp.pad(T.reshape(1, 36), ((0, 0), (0, 28)))

    vals, w8, idx0 = _project(
        x, o_pad, w_pad[:16], w_pad[16:],
        m[:16], sg[:16], gam_pad[:16], bet_pad[:16],
        m[16:], sg[16:], gam_pad[16:], bet_pad[16:], t_flat)

    vals16 = vals.reshape(B * H, F, NPTS)
    out = _scatter(vals16, w8, idx0)
    return out.reshape(B, H * F, G)


# speedup vs baseline: 1.0109x; 1.0109x over previous
import functools

import jax
import jax.numpy as jnp
from jax import lax
from jax.experimental import pallas as pl
from jax.experimental.pallas import tpu as pltpu
from jax.experimental.pallas import tpu_sc as plsc

B = 4
MD = 256
NPTS = 16384
H = 4
F = 64
GS = 16
G = GS ** 3

NB = 2048
OC = H * (F + 3)
OCP = 16 + MD

_PREC = lax.Precision.HIGHEST


def _stats_body(x_ref, s_ref, sx_ref):
    b = pl.program_id(0)
    j = pl.program_id(1)

    @pl.when((b == 0) & (j == 0))
    def _():
        s_ref[...] = jnp.zeros_like(s_ref)
        sx_ref[...] = jnp.zeros_like(sx_ref)

    xb = x_ref[0]
    s_ref[...] += lax.dot_general(xb, xb, (((1,), (1,)), ((), ())),
                                  precision=_PREC,
                                  preferred_element_type=jnp.float32)
    sx_ref[...] += jnp.sum(xb, axis=1, keepdims=True)


def _stats(x):
    return pl.pallas_call(
        _stats_body,
        grid=(B, NPTS // NB),
        in_specs=[pl.BlockSpec((1, MD, NB), lambda b, j: (b, 0, j))],
        out_specs=[
            pl.BlockSpec((MD, MD), lambda b, j: (0, 0)),
            pl.BlockSpec((MD, 1), lambda b, j: (0, 0)),
        ],
        out_shape=[
            jax.ShapeDtypeStruct((MD, MD), jnp.float32),
            jax.ShapeDtypeStruct((MD, 1), jnp.float32),
        ],
    )(x)


def _fold_body(w_ref, s_ref, sx_ref, m_ref, sg_ref):
    w = w_ref[...]
    s = s_ref[...]
    sx = sx_ref[...]
    inv = 1.0 / (B * NPTS)
    m = jnp.dot(w, sx, precision=_PREC,
                preferred_element_type=jnp.float32) * inv
    ws = jnp.dot(w, s, precision=_PREC,
                 preferred_element_type=jnp.float32)
    e2 = jnp.sum(ws * w, axis=1, keepdims=True) * inv
    var = e2 - m * m
    m_ref[...] = m
    sg_ref[...] = jnp.sqrt(var + 1e-5)


def _fold(w_pad, s, sx):
    return pl.pallas_call(
        _fold_body,
        out_shape=[
            jax.ShapeDtypeStruct((OCP, 1), jnp.float32),
            jax.ShapeDtypeStruct((OCP, 1), jnp.float32),
        ],
    )(w_pad, s, sx)


def _proj_body(x_ref, o_ref, wk_ref, wv_ref, mk_ref, sk_ref, gk_ref, bk_ref,
               mv_ref, sv_ref, gv_ref, bv_ref, t_ref,
               val_ref, w8_ref, idx_ref):
    xb = x_ref[0]
    kv_v = jnp.dot(wv_ref[...], xb, preferred_element_type=jnp.float32)
    val_ref[0] = ((kv_v - mv_ref[...]) / sv_ref[...]) * gv_ref[...] + bv_ref[...]
    kv_k = jnp.dot(wk_ref[...], xb, preferred_element_type=jnp.float32)
    k16 = ((kv_k - mk_ref[...]) / sk_ref[...]) * gk_ref[...] + bk_ref[...]
    o = o_ref[0]

    def _bf(v):
        return v.astype(jnp.bfloat16).astype(jnp.float32)

    half = 0.5 * (GS - 1)
    for h in range(H):
        p = [_bf(o[d:d + 1] + k16[3 * h + d:3 * h + d + 1]) for d in range(3)]
        bs, fr = [], []
        for i in range(3):
            key = (_bf(t_ref[0, h * 9 + i * 3 + 0]) * p[0]
                   + _bf(t_ref[0, h * 9 + i * 3 + 1]) * p[1]
                   + _bf(t_ref[0, h * 9 + i * 3 + 2]) * p[2])
            pos = (jnp.tanh(key) + 1.0) * half
            base = jnp.clip(jnp.floor(pos), 0.0, GS - 2.0)
            bs.append(base)
            fr.append(pos - base)
        idx0 = (bs[0].astype(jnp.int32) * (GS * GS)
                + bs[1].astype(jnp.int32) * GS
                + bs[2].astype(jnp.int32))
        idx_ref[0, h] = idx0[0]
        gx = (1.0 - fr[0], fr[0])
        gy = (1.0 - fr[1], fr[1])
        gz = (1.0 - fr[2], fr[2])
        c = 0
        for cx in range(2):
            for cy in range(2):
                for cz in range(2):
                    w8_ref[0, h * 8 + c] = (gx[cx] * gy[cy] * gz[cz])[0]
                    c += 1


def _project(x, o_pad, wk, wv, mk, sk, gk, bk, mv, sv, gv, bv, t_flat):
    return pl.pallas_call(
        _proj_body,
        grid=(B, NPTS // NB),
        in_specs=[
            pl.BlockSpec((1, MD, NB), lambda b, j: (b, 0, j)),
            pl.BlockSpec((1, 8, NB), lambda b, j: (b, 0, j)),
            pl.BlockSpec((16, MD), lambda b, j: (0, 0)),
            pl.BlockSpec((MD, MD), lambda b, j: (0, 0)),
        ] + [pl.BlockSpec((16, 1), lambda b, j: (0, 0))] * 4
          + [pl.BlockSpec((MD, 1), lambda b, j: (0, 0))] * 4
          + [pl.BlockSpec((1, 64), lambda b, j: (0, 0))],
        out_specs=[
            pl.BlockSpec((1, MD, NB), lambda b, j: (b, 0, j)),
            pl.BlockSpec((1, 32, NB), lambda b, j: (b, 0, j)),
            pl.BlockSpec((1, 8, NB), lambda b, j: (b, 0, j)),
        ],
        out_shape=[
            jax.ShapeDtypeStruct((B, MD, NPTS), jnp.float32),
            jax.ShapeDtypeStruct((B, 32, NPTS), jnp.float32),
            jax.ShapeDtypeStruct((B, 8, NPTS), jnp.int32),
        ],
    )(x, o_pad, wk, wv, mk, sk, gk, bk, mv, sv, gv, bv, t_flat)


_NW = 32
_P = 1024
_NCHUNK = NPTS // _P
_OFFS = (0, 1, GS, GS + 1, GS * GS, GS * GS + 1, GS * GS + GS, GS * GS + GS + 1)


def _scatter_body(vals_hbm, w8_hbm, idx_hbm, out_hbm, table,
                  vb0, wb0, ib0, vb1, wb1, ib1, sem0, sem1):
    wid = lax.axis_index("s") * 2 + lax.axis_index("c")
    bufs = ((vb0, wb0, ib0, sem0), (vb1, wb1, ib1, sem1))

    for k in range(2):
        t = wid * 2 + k
        r = t // 4
        q = t % 4
        bb = r // H
        hh = r % H

        @plsc.parallel_loop(0, 16 * G // 16)
        def _zero(i):
            table[pl.ds(i * 16, 16)] = jnp.zeros((16,), jnp.float32)

        def _slices(ch, j):
            p0 = ch * _P
            vb, wb, ib, sem = bufs[j]
            return (
                (vals_hbm.at[r, pl.ds(q * 16, 16), pl.ds(p0, _P)], vb, sem),
                (w8_hbm.at[bb, pl.ds(hh * 8, 8), pl.ds(p0, _P)], wb, sem),
                (idx_hbm.at[bb, hh, pl.ds(p0, _P)], ib, sem),
            )

        def _issue(ch, j):
            for src, dst, sem in _slices(ch, j):
                pltpu.async_copy(src, dst, sem)

        def _wait(ch, j):
            for src, dst, sem in _slices(ch, j):
                pltpu.make_async_copy(src, dst, sem).wait()

        def _compute(j):
            vb, wb, ib, _ = bufs[j]

            @plsc.parallel_loop(0, _P // 16)
            def _group(g):
                s = g * 16
                idxv = ib[pl.ds(s, 16)]
                wcs = [wb[c, pl.ds(s, 16)] for c in range(8)]
                acs = [idxv + _OFFS[c] for c in range(8)]
                for f in range(16):
                    v = vb[f, pl.ds(s, 16)]
                    fG = f * G
                    for c in range(8):
                        plsc.addupdate_scatter(table, [acs[c] + fG],
                                               v * wcs[c])

        _issue(0, 0)
        _issue(1, 1)

        def _pair(cp, _):
            ch0 = cp * 2
            _wait(ch0, 0)
            _compute(0)

            @pl.when(ch0 + 2 < _NCHUNK)
            def _():
                _issue(ch0 + 2, 0)

            _wait(ch0 + 1, 1)
            _compute(1)

            @pl.when(ch0 + 3 < _NCHUNK)
            def _():
                _issue(ch0 + 3, 1)
            return 0

        lax.fori_loop(0, _NCHUNK // 2, _pair, 0)

        for f in range(16):
            pltpu.sync_copy(table.at[pl.ds(f * G, G)],
                            out_hbm.at[r, q * 16 + f, :])


def _scatter(vals16, w8, idx0):
    mesh = plsc.VectorSubcoreMesh(core_axis_name="c", subcore_axis_name="s")
    f = pl.kernel(
        _scatter_body,
        out_type=jax.ShapeDtypeStruct((B * H, F, G), jnp.float32),
        mesh=mesh,
        compiler_params=pltpu.CompilerParams(needs_layout_passes=False),
        scratch_types=[
            pltpu.VMEM((16 * G,), jnp.float32),
            pltpu.VMEM((16, _P), jnp.float32),
            pltpu.VMEM((8, _P), jnp.float32),
            pltpu.VMEM((_P,), jnp.int32),
            pltpu.VMEM((16, _P), jnp.float32),
            pltpu.VMEM((8, _P), jnp.float32),
            pltpu.VMEM((_P,), jnp.int32),
            pltpu.SemaphoreType.DMA,
            pltpu.SemaphoreType.DMA,
        ],
    )
    return f(vals16, w8, idx0)


def kernel(input_tensor, original_points, W_kv, gamma_k, beta_k,
           gamma_v, beta_v, T):
    x = input_tensor.astype(jnp.float32)

    w_pad = jnp.concatenate(
        [W_kv[:H * 3], jnp.zeros((4, MD), jnp.float32), W_kv[H * 3:]], axis=0)
    gam_pad = jnp.concatenate(
        [gamma_k, jnp.zeros((4,), jnp.float32), gamma_v])[:, None]
    bet_pad = jnp.concatenate(
        [beta_k, jnp.zeros((4,), jnp.float32), beta_v])[:, None]

    s, sx = _stats(x)
    m, sg = _fold(w_pad, s, sx)

    o_pad = jnp.concatenate(
        [original_points, jnp.zeros((B, 5, NPTS), jnp.float32)], axis=1)
    t_flat = jnp.pad(T.reshape(1, 36), ((0, 0), (0, 28)))

    vals, w8, idx0 = _project(
        x, o_pad, w_pad[:16], w_pad[16:],
        m[:16], sg[:16], gam_pad[:16], bet_pad[:16],
        m[16:], sg[16:], gam_pad[16:], bet_pad[16:], t_flat)

    vals16 = vals.reshape(B * H, F, NPTS)
    out = _scatter(vals16, w8, idx0)
    return out.reshape(B, H * F, G)
